# baseline (speedup 1.0000x reference)
import functools

import numpy as np

import jax
import jax.numpy as jnp
from jax import lax
from jax.experimental import pallas as pl
from jax.experimental.pallas import tpu as pltpu
from jax.experimental.pallas import tpu_sc as plsc

_ANCHOR_SIZES = [64.0, 128.0, 256.0]
_ANCHOR_RATIOS = [float(np.sqrt(r)) for r in [0.5, 1.0, 2.0]]
_ANCHORS = np.array(
    [[s * r, s / r] for s in _ANCHOR_SIZES for r in _ANCHOR_RATIOS],
    dtype=np.float32,
)

_PRE_NMS = 6000
_POST_NMS = 300
_IOU_THR = 0.7
_NEG = -1e9
_BIG_IDX = 1 << 30
_N = 9216
_ROWS = 72
_CROWS = 48
_NC = 6144
_LANES = 128
_B = 4


def _iota2(shape, dim):
    return jax.lax.broadcasted_iota(jnp.int32, shape, dim)


def _redmax(x):
    return jnp.max(jnp.max(x, axis=2, keepdims=True), axis=1, keepdims=True)


def _redmin(x):
    return jnp.min(jnp.min(x, axis=2, keepdims=True), axis=1, keepdims=True)


def _redsum(x):
    return jnp.sum(jnp.sum(x, axis=2, keepdims=True), axis=1, keepdims=True)


def _prep_body(s_ref, tx_ref, ty_ref, tw_ref, th_ref,
               act_ref, pos_ref, x1_ref, y1_ref, x2_ref, y2_ref):
    s = s_ref[...]
    shape3 = s.shape

    flat = _iota2((_ROWS, _LANES), 0) * _LANES + _iota2((_ROWS, _LANES), 1)
    a_idx = flat >> 10
    hw = flat & 1023
    hh = (hw >> 5).astype(jnp.float32)
    ww = (hw & 31).astype(jnp.float32)

    wa = jnp.zeros((_ROWS, _LANES), jnp.float32)
    ha = jnp.zeros((_ROWS, _LANES), jnp.float32)
    for k in range(9):
        sel = a_idx == k
        wa = jnp.where(sel, jnp.float32(_ANCHORS[k, 0]), wa)
        ha = jnp.where(sel, jnp.float32(_ANCHORS[k, 1]), ha)

    px = (ww + 0.5) * 16.0
    py = (hh + 0.5) * 16.0
    ax1 = px - wa / 2.0
    ay1 = py - ha / 2.0
    cx = ax1 + 0.5 * wa
    cy = ay1 + 0.5 * ha

    ncx = cx + tx_ref[...] * wa
    ncy = cy + ty_ref[...] * ha
    nw = wa * jnp.exp(tw_ref[...])
    nh = ha * jnp.exp(th_ref[...])
    x1_ref[...] = jnp.clip(ncx - 0.5 * nw, 0.0, 511.0)
    y1_ref[...] = jnp.clip(ncy - 0.5 * nh, 0.0, 511.0)
    x2_ref[...] = jnp.clip(ncx + 0.5 * nw, 0.0, 511.0)
    y2_ref[...] = jnp.clip(ncy + 0.5 * nh, 0.0, 511.0)

    s_bits = jax.lax.bitcast_convert_type(s, jnp.int32)

    def bs_step(_, carry):
        lo, hi = carry
        mid = (lo + hi) >> 1
        cnt = _redsum(jnp.where(s_bits > mid, 1.0, 0.0))
        pred = cnt < float(_PRE_NMS)
        return jnp.where(pred, lo, mid + 1), jnp.where(pred, mid, hi)

    lo0 = jnp.zeros((_B, 1, 1), jnp.int32)
    hi0 = jnp.full((_B, 1, 1), 0x3F800000, jnp.int32)
    lo_f, _ = jax.lax.fori_loop(0, 31, bs_step, (lo0, hi0))
    thr = jax.lax.bitcast_convert_type(lo_f, jnp.float32)

    gt = s > thr
    eq = s == thr
    cg = _redsum(jnp.where(gt, 1.0, 0.0))
    r_adm = float(_PRE_NMS) - cg

    eqf = jnp.where(eq, 1.0, 0.0).reshape(_B * _ROWS, _LANES)
    lane_lt = jnp.where(
        _iota2((_LANES, _LANES), 0) < _iota2((_LANES, _LANES), 1), 1.0, 0.0)
    in_row = jax.lax.dot(eqf, lane_lt,
                         precision=jax.lax.Precision.HIGHEST,
                         preferred_element_type=jnp.float32)
    rowsum = jnp.sum(eqf, axis=1, keepdims=True)
    p = _iota2((_B * _ROWS, _B * _ROWS), 0)
    q = _iota2((_B * _ROWS, _B * _ROWS), 1)
    row_lt = jnp.where(((p // _ROWS) == (q // _ROWS)) & (q < p), 1.0, 0.0)
    row_off = jax.lax.dot(row_lt, rowsum,
                          precision=jax.lax.Precision.HIGHEST,
                          preferred_element_type=jnp.float32)
    prefix = (in_row + row_off).reshape(shape3)

    member = gt | (eq & (prefix < r_adm))
    act_ref[...] = jnp.where(member, s, _NEG)

    mf = jnp.where(member, 1.0, 0.0).reshape(_B * _ROWS, _LANES)
    m_in_row = jax.lax.dot(mf, lane_lt,
                           precision=jax.lax.Precision.HIGHEST,
                           preferred_element_type=jnp.float32)
    m_rowsum = jnp.sum(mf, axis=1, keepdims=True)
    m_row_off = jax.lax.dot(row_lt, m_rowsum,
                            precision=jax.lax.Precision.HIGHEST,
                            preferred_element_type=jnp.float32)
    pos_ref[...] = (m_in_row + m_row_off).reshape(shape3).astype(jnp.int32)


_SC_CHUNKS = _N // 16


def _compact_sc(act, pos, x1, y1, x2, y2):
    mesh = plsc.VectorSubcoreMesh(core_axis_name="c", subcore_axis_name="s")
    out_type = tuple(jax.ShapeDtypeStruct((_B, _NC), jnp.float32)
                     for _ in range(5))
    scratch = ([pltpu.VMEM((_N,), jnp.float32) for _ in range(5)]
               + [pltpu.VMEM((_N,), jnp.int32)]
               + [pltpu.VMEM((_NC,), jnp.float32) for _ in range(5)])

    @functools.partial(
        pl.kernel, out_type=out_type, mesh=mesh, scratch_types=scratch,
        compiler_params=pltpu.CompilerParams(needs_layout_passes=False))
    def k(act_h, pos_h, x1_h, y1_h, x2_h, y2_h,
          oa_h, o1_h, o2_h, o3_h, o4_h,
          av, v1, v2, v3, v4, pv, oa, o1, o2, o3, o4):
        wid = lax.axis_index("s") * 2 + lax.axis_index("c")

        @pl.when(wid < _B)
        def _():
            pltpu.sync_copy(act_h.at[wid], av)
            pltpu.sync_copy(pos_h.at[wid], pv)
            pltpu.sync_copy(x1_h.at[wid], v1)
            pltpu.sync_copy(y1_h.at[wid], v2)
            pltpu.sync_copy(x2_h.at[wid], v3)
            pltpu.sync_copy(y2_h.at[wid], v4)

            def chunk(i, _):
                base = i * 16
                s16 = av[pl.ds(base, 16)]
                m = s16 > jnp.float32(-1e8)
                posv = pv[pl.ds(base, 16)]
                plsc.store_scatter(oa, [posv], s16, mask=m)
                plsc.store_scatter(o1, [posv], v1[pl.ds(base, 16)], mask=m)
                plsc.store_scatter(o2, [posv], v2[pl.ds(base, 16)], mask=m)
                plsc.store_scatter(o3, [posv], v3[pl.ds(base, 16)], mask=m)
                plsc.store_scatter(o4, [posv], v4[pl.ds(base, 16)], mask=m)
                return 0

            lax.fori_loop(0, _SC_CHUNKS, chunk, 0, unroll=False)

            negv = jnp.full((16,), _NEG, jnp.float32)
            zerov = jnp.zeros((16,), jnp.float32)
            for j in range((_NC - _PRE_NMS) // 16):
                base = _PRE_NMS + j * 16
                oa[pl.ds(base, 16)] = negv
                o1[pl.ds(base, 16)] = zerov
                o2[pl.ds(base, 16)] = zerov
                o3[pl.ds(base, 16)] = zerov
                o4[pl.ds(base, 16)] = zerov

            pltpu.sync_copy(oa, oa_h.at[wid])
            pltpu.sync_copy(o1, o1_h.at[wid])
            pltpu.sync_copy(o2, o2_h.at[wid])
            pltpu.sync_copy(o3, o3_h.at[wid])
            pltpu.sync_copy(o4, o4_h.at[wid])

    return k(act, pos, x1, y1, x2, y2)


def _scan_body(act_ref, x1_ref, y1_ref, x2_ref, y2_ref, out_ref):
    act0 = act_ref[...]
    bx1 = x1_ref[...]
    by1 = y1_ref[...]
    bx2 = x2_ref[...]
    by2 = y2_ref[...]
    area = jnp.maximum(bx2 - bx1, 0.0) * jnp.maximum(by2 - by1, 0.0)
    flat = _iota2((_CROWS, _LANES), 0) * _LANES + _iota2((_CROWS, _LANES), 1)

    m0 = _redmax(act0)
    i0 = _redmin(jnp.where(act0 == m0, flat, _BIG_IDX))

    def step(t, active):
        m = _redmax(active)
        wi_raw = _redmin(jnp.where(active == m, flat, _BIG_IDX))
        wi = jnp.where(m > _NEG, wi_raw, i0)
        onehot = flat == wi
        wx1 = _redmax(jnp.where(onehot, bx1, _NEG))
        wy1 = _redmax(jnp.where(onehot, by1, _NEG))
        wx2 = _redmax(jnp.where(onehot, bx2, _NEG))
        wy2 = _redmax(jnp.where(onehot, by2, _NEG))

        xx1 = jnp.maximum(wx1, bx1)
        yy1 = jnp.maximum(wy1, by1)
        xx2 = jnp.minimum(wx2, bx2)
        yy2 = jnp.minimum(wy2, by2)
        inter = jnp.maximum(xx2 - xx1, 0.0) * jnp.maximum(yy2 - yy1, 0.0)
        wa1 = jnp.maximum(wx2 - wx1, 0.0) * jnp.maximum(wy2 - wy1, 0.0)
        iou = inter / (wa1 + area - inter + 1e-8)

        new_active = jnp.where(iou > _IOU_THR, _NEG, active)
        new_active = jnp.where(onehot, _NEG, new_active)

        row = jnp.concatenate([wx1, wy1, wx2, wy2], axis=2)
        out_ref[t] = row.reshape(_B, 4)
        return new_active

    jax.lax.fori_loop(0, _POST_NMS, step, act0)


def kernel(rpn_scores, rpn_deltas, input_image):
    del input_image
    s = rpn_scores.reshape(_B, _ROWS, _LANES)
    tx = rpn_deltas[:, 0::4].reshape(_B, _ROWS, _LANES)
    ty = rpn_deltas[:, 1::4].reshape(_B, _ROWS, _LANES)
    tw = rpn_deltas[:, 2::4].reshape(_B, _ROWS, _LANES)
    th = rpn_deltas[:, 3::4].reshape(_B, _ROWS, _LANES)

    shp = jax.ShapeDtypeStruct((_B, _ROWS, _LANES), jnp.float32)
    shp_i = jax.ShapeDtypeStruct((_B, _ROWS, _LANES), jnp.int32)
    act, pos, x1, y1, x2, y2 = pl.pallas_call(
        _prep_body,
        out_shape=(shp, shp_i, shp, shp, shp, shp),
    )(s, tx, ty, tw, th)

    flat2 = lambda a: a.reshape(_B, _N)
    act_c, x1_c, y1_c, x2_c, y2_c = _compact_sc(
        flat2(act), flat2(pos), flat2(x1), flat2(y1), flat2(x2), flat2(y2))

    r3 = lambda a: a.reshape(_B, _CROWS, _LANES)
    out = pl.pallas_call(
        _scan_body,
        out_shape=jax.ShapeDtypeStruct((_POST_NMS, _B, 4), jnp.float32),
    )(r3(act_c), r3(x1_c), r3(y1_c), r3(x2_c), r3(y2_c))
    return out.transpose(1, 0, 2)


# speedup vs baseline: 1.0009x; 1.0009x over previous
import functools

import numpy as np

import jax
import jax.numpy as jnp
from jax import lax
from jax.experimental import pallas as pl
from jax.experimental.pallas import tpu as pltpu
from jax.experimental.pallas import tpu_sc as plsc

_ANCHOR_SIZES = [64.0, 128.0, 256.0]
_ANCHOR_RATIOS = [float(np.sqrt(r)) for r in [0.5, 1.0, 2.0]]
_ANCHORS = np.array(
    [[s * r, s / r] for s in _ANCHOR_SIZES for r in _ANCHOR_RATIOS],
    dtype=np.float32,
)

_PRE_NMS = 6000
_POST_NMS = 300
_IOU_THR = 0.7
_NEG = -1e9
_BIG_IDX = 1 << 30
_N = 9216
_ROWS = 72
_CROWS = 48
_NC = 6144
_LANES = 128
_B = 4


def _iota2(shape, dim):
    return jax.lax.broadcasted_iota(jnp.int32, shape, dim)


def _redmax(x):
    return jnp.max(jnp.max(x, axis=2, keepdims=True), axis=1, keepdims=True)


def _redmin(x):
    return jnp.min(jnp.min(x, axis=2, keepdims=True), axis=1, keepdims=True)


def _redsum(x):
    return jnp.sum(jnp.sum(x, axis=2, keepdims=True), axis=1, keepdims=True)


def _prep_body(s_ref, tx_ref, ty_ref, tw_ref, th_ref,
               act_ref, pos_ref, x1_ref, y1_ref, x2_ref, y2_ref):
    s = s_ref[...]
    shape3 = s.shape

    flat = _iota2((_ROWS, _LANES), 0) * _LANES + _iota2((_ROWS, _LANES), 1)
    a_idx = flat >> 10
    hw = flat & 1023
    hh = (hw >> 5).astype(jnp.float32)
    ww = (hw & 31).astype(jnp.float32)

    wa = jnp.zeros((_ROWS, _LANES), jnp.float32)
    ha = jnp.zeros((_ROWS, _LANES), jnp.float32)
    for k in range(9):
        sel = a_idx == k
        wa = jnp.where(sel, jnp.float32(_ANCHORS[k, 0]), wa)
        ha = jnp.where(sel, jnp.float32(_ANCHORS[k, 1]), ha)

    px = (ww + 0.5) * 16.0
    py = (hh + 0.5) * 16.0
    ax1 = px - wa / 2.0
    ay1 = py - ha / 2.0
    cx = ax1 + 0.5 * wa
    cy = ay1 + 0.5 * ha

    ncx = cx + tx_ref[...] * wa
    ncy = cy + ty_ref[...] * ha
    nw = wa * jnp.exp(tw_ref[...])
    nh = ha * jnp.exp(th_ref[...])
    x1_ref[...] = jnp.clip(ncx - 0.5 * nw, 0.0, 511.0)
    y1_ref[...] = jnp.clip(ncy - 0.5 * nh, 0.0, 511.0)
    x2_ref[...] = jnp.clip(ncx + 0.5 * nw, 0.0, 511.0)
    y2_ref[...] = jnp.clip(ncy + 0.5 * nh, 0.0, 511.0)

    s_bits = jax.lax.bitcast_convert_type(s, jnp.int32)

    def bs_step(_, carry):
        lo, hi = carry
        mid = (lo + hi) >> 1
        cnt = _redsum(jnp.where(s_bits > mid, 1.0, 0.0))
        pred = cnt < float(_PRE_NMS)
        return jnp.where(pred, lo, mid + 1), jnp.where(pred, mid, hi)

    lo0 = jnp.zeros((_B, 1, 1), jnp.int32)
    hi0 = jnp.full((_B, 1, 1), 0x3F800000, jnp.int32)
    lo_f, _ = jax.lax.fori_loop(0, 31, bs_step, (lo0, hi0))
    thr = jax.lax.bitcast_convert_type(lo_f, jnp.float32)

    gt = s > thr
    eq = s == thr
    cg = _redsum(jnp.where(gt, 1.0, 0.0))
    r_adm = float(_PRE_NMS) - cg

    eqf = jnp.where(eq, 1.0, 0.0).reshape(_B * _ROWS, _LANES)
    lane_lt = jnp.where(
        _iota2((_LANES, _LANES), 0) < _iota2((_LANES, _LANES), 1), 1.0, 0.0)
    in_row = jax.lax.dot(eqf, lane_lt,
                         precision=jax.lax.Precision.HIGHEST,
                         preferred_element_type=jnp.float32)
    rowsum = jnp.sum(eqf, axis=1, keepdims=True)
    p = _iota2((_B * _ROWS, _B * _ROWS), 0)
    q = _iota2((_B * _ROWS, _B * _ROWS), 1)
    row_lt = jnp.where(((p // _ROWS) == (q // _ROWS)) & (q < p), 1.0, 0.0)
    row_off = jax.lax.dot(row_lt, rowsum,
                          precision=jax.lax.Precision.HIGHEST,
                          preferred_element_type=jnp.float32)
    prefix = (in_row + row_off).reshape(shape3)

    member = gt | (eq & (prefix < r_adm))
    act_ref[...] = jnp.where(member, s, _NEG)

    mf = jnp.where(member, 1.0, 0.0).reshape(_B * _ROWS, _LANES)
    m_in_row = jax.lax.dot(mf, lane_lt,
                           precision=jax.lax.Precision.HIGHEST,
                           preferred_element_type=jnp.float32)
    m_rowsum = jnp.sum(mf, axis=1, keepdims=True)
    m_row_off = jax.lax.dot(row_lt, m_rowsum,
                            precision=jax.lax.Precision.HIGHEST,
                            preferred_element_type=jnp.float32)
    pos_ref[...] = (m_in_row + m_row_off).reshape(shape3).astype(jnp.int32)


_SC_CHUNKS = _N // 16


def _compact_sc(act, pos, x1, y1, x2, y2):
    mesh = plsc.VectorSubcoreMesh(core_axis_name="c", subcore_axis_name="s")
    out_type = tuple(jax.ShapeDtypeStruct((_B, _NC), jnp.float32)
                     for _ in range(5))
    scratch = ([pltpu.VMEM((_N,), jnp.float32) for _ in range(5)]
               + [pltpu.VMEM((_N,), jnp.int32)]
               + [pltpu.VMEM((_NC,), jnp.float32) for _ in range(5)])

    @functools.partial(
        pl.kernel, out_type=out_type, mesh=mesh, scratch_types=scratch,
        compiler_params=pltpu.CompilerParams(needs_layout_passes=False))
    def k(act_h, pos_h, x1_h, y1_h, x2_h, y2_h,
          oa_h, o1_h, o2_h, o3_h, o4_h,
          av, v1, v2, v3, v4, pv, oa, o1, o2, o3, o4):
        wid = lax.axis_index("s") * 2 + lax.axis_index("c")

        @pl.when(wid < _B)
        def _():
            pltpu.sync_copy(act_h.at[wid], av)
            pltpu.sync_copy(pos_h.at[wid], pv)
            pltpu.sync_copy(x1_h.at[wid], v1)
            pltpu.sync_copy(y1_h.at[wid], v2)
            pltpu.sync_copy(x2_h.at[wid], v3)
            pltpu.sync_copy(y2_h.at[wid], v4)

            def chunk(i, _):
                base = i * 16
                s16 = av[pl.ds(base, 16)]
                m = s16 > jnp.float32(-1e8)
                posv = pv[pl.ds(base, 16)]
                plsc.store_scatter(oa, [posv], s16, mask=m)
                plsc.store_scatter(o1, [posv], v1[pl.ds(base, 16)], mask=m)
                plsc.store_scatter(o2, [posv], v2[pl.ds(base, 16)], mask=m)
                plsc.store_scatter(o3, [posv], v3[pl.ds(base, 16)], mask=m)
                plsc.store_scatter(o4, [posv], v4[pl.ds(base, 16)], mask=m)
                return 0

            lax.fori_loop(0, _SC_CHUNKS, chunk, 0, unroll=False)

            negv = jnp.full((16,), _NEG, jnp.float32)
            zerov = jnp.zeros((16,), jnp.float32)
            for j in range((_NC - _PRE_NMS) // 16):
                base = _PRE_NMS + j * 16
                oa[pl.ds(base, 16)] = negv
                o1[pl.ds(base, 16)] = zerov
                o2[pl.ds(base, 16)] = zerov
                o3[pl.ds(base, 16)] = zerov
                o4[pl.ds(base, 16)] = zerov

            pltpu.sync_copy(oa, oa_h.at[wid])
            pltpu.sync_copy(o1, o1_h.at[wid])
            pltpu.sync_copy(o2, o2_h.at[wid])
            pltpu.sync_copy(o3, o3_h.at[wid])
            pltpu.sync_copy(o4, o4_h.at[wid])

    return k(act, pos, x1, y1, x2, y2)


def _scan_body(act_ref, x1_ref, y1_ref, x2_ref, y2_ref, out_ref):
    act0 = act_ref[...]
    bx1 = x1_ref[...]
    by1 = y1_ref[...]
    bx2 = x2_ref[...]
    by2 = y2_ref[...]
    area = jnp.maximum(bx2 - bx1, 0.0) * jnp.maximum(by2 - by1, 0.0)
    flat = _iota2((_CROWS, _LANES), 0) * _LANES + _iota2((_CROWS, _LANES), 1)

    m0 = _redmax(act0)
    i0 = _redmin(jnp.where(act0 == m0, flat, _BIG_IDX))

    def step(t, active):
        m = _redmax(active)
        wi_raw = _redmin(jnp.where(active == m, flat, _BIG_IDX))
        wi = jnp.where(m > _NEG, wi_raw, i0)
        onehot = flat == wi
        stacked = jnp.concatenate(
            [jnp.where(onehot, c, _NEG) for c in (bx1, by1, bx2, by2)],
            axis=0)
        w = _redmax(stacked)
        wx1 = w[0:_B]
        wy1 = w[_B:2 * _B]
        wx2 = w[2 * _B:3 * _B]
        wy2 = w[3 * _B:]

        xx1 = jnp.maximum(wx1, bx1)
        yy1 = jnp.maximum(wy1, by1)
        xx2 = jnp.minimum(wx2, bx2)
        yy2 = jnp.minimum(wy2, by2)
        inter = jnp.maximum(xx2 - xx1, 0.0) * jnp.maximum(yy2 - yy1, 0.0)
        wa1 = jnp.maximum(wx2 - wx1, 0.0) * jnp.maximum(wy2 - wy1, 0.0)
        iou = inter / (wa1 + area - inter + 1e-8)

        new_active = jnp.where((iou > _IOU_THR) | onehot, _NEG, active)

        row = jnp.concatenate([wx1, wy1, wx2, wy2], axis=2)
        out_ref[t] = row.reshape(_B, 4)
        return new_active

    jax.lax.fori_loop(0, _POST_NMS, step, act0)


def kernel(rpn_scores, rpn_deltas, input_image):
    del input_image
    s = rpn_scores.reshape(_B, _ROWS, _LANES)
    tx = rpn_deltas[:, 0::4].reshape(_B, _ROWS, _LANES)
    ty = rpn_deltas[:, 1::4].reshape(_B, _ROWS, _LANES)
    tw = rpn_deltas[:, 2::4].reshape(_B, _ROWS, _LANES)
    th = rpn_deltas[:, 3::4].reshape(_B, _ROWS, _LANES)

    shp = jax.ShapeDtypeStruct((_B, _ROWS, _LANES), jnp.float32)
    shp_i = jax.ShapeDtypeStruct((_B, _ROWS, _LANES), jnp.int32)
    act, pos, x1, y1, x2, y2 = pl.pallas_call(
        _prep_body,
        out_shape=(shp, shp_i, shp, shp, shp, shp),
    )(s, tx, ty, tw, th)

    flat2 = lambda a: a.reshape(_B, _N)
    act_c, x1_c, y1_c, x2_c, y2_c = _compact_sc(
        flat2(act), flat2(pos), flat2(x1), flat2(y1), flat2(x2), flat2(y2))

    r3 = lambda a: a.reshape(_B, _CROWS, _LANES)
    out = pl.pallas_call(
        _scan_body,
        out_shape=jax.ShapeDtypeStruct((_POST_NMS, _B, 4), jnp.float32),
    )(r3(act_c), r3(x1_c), r3(y1_c), r3(x2_c), r3(y2_c))
    return out.transpose(1, 0, 2)


# speedup vs baseline: 1.2446x; 1.2435x over previous
import functools

import numpy as np

import jax
import jax.numpy as jnp
from jax import lax
from jax.experimental import pallas as pl
from jax.experimental.pallas import tpu as pltpu
from jax.experimental.pallas import tpu_sc as plsc

_ANCHOR_SIZES = [64.0, 128.0, 256.0]
_ANCHOR_RATIOS = [float(np.sqrt(r)) for r in [0.5, 1.0, 2.0]]
_ANCHORS = np.array(
    [[s * r, s / r] for s in _ANCHOR_SIZES for r in _ANCHOR_RATIOS],
    dtype=np.float32,
)

_PRE_NMS = 6000
_POST_NMS = 300
_IOU_THR = 0.7
_NEG = -1e9
_BIG_IDX = 1 << 30
_N = 9216
_ROWS = 72
_CROWS = 48
_NC = 6144
_LANES = 128
_B = 4


def _iota2(shape, dim):
    return jax.lax.broadcasted_iota(jnp.int32, shape, dim)


def _redmax(x):
    return jnp.max(jnp.max(x, axis=1, keepdims=True), axis=2, keepdims=True)


def _redmin(x):
    return jnp.min(jnp.min(x, axis=1, keepdims=True), axis=2, keepdims=True)


def _redsum(x):
    return jnp.sum(jnp.sum(x, axis=1, keepdims=True), axis=2, keepdims=True)


def _prep_body(s_ref, tx_ref, ty_ref, tw_ref, th_ref,
               act_ref, pos_ref, x1_ref, y1_ref, x2_ref, y2_ref):
    s = s_ref[...]
    shape3 = s.shape

    flat = _iota2((_ROWS, _LANES), 0) * _LANES + _iota2((_ROWS, _LANES), 1)
    a_idx = flat >> 10
    hw = flat & 1023
    hh = (hw >> 5).astype(jnp.float32)
    ww = (hw & 31).astype(jnp.float32)

    wa = jnp.zeros((_ROWS, _LANES), jnp.float32)
    ha = jnp.zeros((_ROWS, _LANES), jnp.float32)
    for k in range(9):
        sel = a_idx == k
        wa = jnp.where(sel, jnp.float32(_ANCHORS[k, 0]), wa)
        ha = jnp.where(sel, jnp.float32(_ANCHORS[k, 1]), ha)

    px = (ww + 0.5) * 16.0
    py = (hh + 0.5) * 16.0
    ax1 = px - wa / 2.0
    ay1 = py - ha / 2.0
    cx = ax1 + 0.5 * wa
    cy = ay1 + 0.5 * ha

    ncx = cx + tx_ref[...] * wa
    ncy = cy + ty_ref[...] * ha
    nw = wa * jnp.exp(tw_ref[...])
    nh = ha * jnp.exp(th_ref[...])
    x1_ref[...] = jnp.clip(ncx - 0.5 * nw, 0.0, 511.0)
    y1_ref[...] = jnp.clip(ncy - 0.5 * nh, 0.0, 511.0)
    x2_ref[...] = jnp.clip(ncx + 0.5 * nw, 0.0, 511.0)
    y2_ref[...] = jnp.clip(ncy + 0.5 * nh, 0.0, 511.0)

    s_bits = jax.lax.bitcast_convert_type(s, jnp.int32)

    def bs_step(_, carry):
        lo, hi = carry
        mid = (lo + hi) >> 1
        cnt = _redsum(jnp.where(s_bits > mid, 1.0, 0.0))
        pred = cnt < float(_PRE_NMS)
        return jnp.where(pred, lo, mid + 1), jnp.where(pred, mid, hi)

    lo0 = jnp.zeros((_B, 1, 1), jnp.int32)
    hi0 = jnp.full((_B, 1, 1), 0x3F800000, jnp.int32)
    lo_f, _ = jax.lax.fori_loop(0, 31, bs_step, (lo0, hi0))
    thr = jax.lax.bitcast_convert_type(lo_f, jnp.float32)

    gt = s > thr
    eq = s == thr
    cg = _redsum(jnp.where(gt, 1.0, 0.0))
    r_adm = float(_PRE_NMS) - cg

    eqf = jnp.where(eq, 1.0, 0.0).reshape(_B * _ROWS, _LANES)
    lane_lt = jnp.where(
        _iota2((_LANES, _LANES), 0) < _iota2((_LANES, _LANES), 1), 1.0, 0.0)
    in_row = jax.lax.dot(eqf, lane_lt,
                         precision=jax.lax.Precision.HIGHEST,
                         preferred_element_type=jnp.float32)
    rowsum = jnp.sum(eqf, axis=1, keepdims=True)
    p = _iota2((_B * _ROWS, _B * _ROWS), 0)
    q = _iota2((_B * _ROWS, _B * _ROWS), 1)
    row_lt = jnp.where(((p // _ROWS) == (q // _ROWS)) & (q < p), 1.0, 0.0)
    row_off = jax.lax.dot(row_lt, rowsum,
                          precision=jax.lax.Precision.HIGHEST,
                          preferred_element_type=jnp.float32)
    prefix = (in_row + row_off).reshape(shape3)

    member = gt | (eq & (prefix < r_adm))
    act_ref[...] = jnp.where(member, s, _NEG)

    mf = jnp.where(member, 1.0, 0.0).reshape(_B * _ROWS, _LANES)
    m_in_row = jax.lax.dot(mf, lane_lt,
                           precision=jax.lax.Precision.HIGHEST,
                           preferred_element_type=jnp.float32)
    m_rowsum = jnp.sum(mf, axis=1, keepdims=True)
    m_row_off = jax.lax.dot(row_lt, m_rowsum,
                            precision=jax.lax.Precision.HIGHEST,
                            preferred_element_type=jnp.float32)
    pos_ref[...] = (m_in_row + m_row_off).reshape(shape3).astype(jnp.int32)


_SC_CHUNKS = _N // 16


def _compact_sc(act, pos, x1, y1, x2, y2):
    mesh = plsc.VectorSubcoreMesh(core_axis_name="c", subcore_axis_name="s")
    out_type = tuple(jax.ShapeDtypeStruct((_B, _NC), jnp.float32)
                     for _ in range(5))
    scratch = ([pltpu.VMEM((_N,), jnp.float32) for _ in range(5)]
               + [pltpu.VMEM((_N,), jnp.int32)]
               + [pltpu.VMEM((_NC,), jnp.float32) for _ in range(5)])

    @functools.partial(
        pl.kernel, out_type=out_type, mesh=mesh, scratch_types=scratch,
        compiler_params=pltpu.CompilerParams(needs_layout_passes=False))
    def k(act_h, pos_h, x1_h, y1_h, x2_h, y2_h,
          oa_h, o1_h, o2_h, o3_h, o4_h,
          av, v1, v2, v3, v4, pv, oa, o1, o2, o3, o4):
        wid = lax.axis_index("s") * 2 + lax.axis_index("c")

        @pl.when(wid < _B)
        def _():
            pltpu.sync_copy(act_h.at[wid], av)
            pltpu.sync_copy(pos_h.at[wid], pv)
            pltpu.sync_copy(x1_h.at[wid], v1)
            pltpu.sync_copy(y1_h.at[wid], v2)
            pltpu.sync_copy(x2_h.at[wid], v3)
            pltpu.sync_copy(y2_h.at[wid], v4)

            def chunk(i, _):
                base = i * 16
                s16 = av[pl.ds(base, 16)]
                m = s16 > jnp.float32(-1e8)
                posv = pv[pl.ds(base, 16)]
                plsc.store_scatter(oa, [posv], s16, mask=m)
                plsc.store_scatter(o1, [posv], v1[pl.ds(base, 16)], mask=m)
                plsc.store_scatter(o2, [posv], v2[pl.ds(base, 16)], mask=m)
                plsc.store_scatter(o3, [posv], v3[pl.ds(base, 16)], mask=m)
                plsc.store_scatter(o4, [posv], v4[pl.ds(base, 16)], mask=m)
                return 0

            lax.fori_loop(0, _SC_CHUNKS, chunk, 0, unroll=False)

            negv = jnp.full((16,), _NEG, jnp.float32)
            zerov = jnp.zeros((16,), jnp.float32)
            for j in range((_NC - _PRE_NMS) // 16):
                base = _PRE_NMS + j * 16
                oa[pl.ds(base, 16)] = negv
                o1[pl.ds(base, 16)] = zerov
                o2[pl.ds(base, 16)] = zerov
                o3[pl.ds(base, 16)] = zerov
                o4[pl.ds(base, 16)] = zerov

            pltpu.sync_copy(oa, oa_h.at[wid])
            pltpu.sync_copy(o1, o1_h.at[wid])
            pltpu.sync_copy(o2, o2_h.at[wid])
            pltpu.sync_copy(o3, o3_h.at[wid])
            pltpu.sync_copy(o4, o4_h.at[wid])

    return k(act, pos, x1, y1, x2, y2)


def _scan_body(act_ref, x1_ref, y1_ref, x2_ref, y2_ref, out_ref):
    act0 = act_ref[...]
    bx1 = x1_ref[...]
    by1 = y1_ref[...]
    bx2 = x2_ref[...]
    by2 = y2_ref[...]
    area = jnp.maximum(bx2 - bx1, 0.0) * jnp.maximum(by2 - by1, 0.0)
    flat = (_iota2((_CROWS, _LANES), 0) * _LANES
            + _iota2((_CROWS, _LANES), 1)).astype(jnp.float32)

    m0 = _redmax(act0)
    i0 = _redmin(jnp.where(act0 == m0, flat, jnp.float32(_BIG_IDX)))

    def step(t, active):
        m = _redmax(active)
        wi_raw = _redmin(jnp.where(active == m, flat, jnp.float32(_BIG_IDX)))
        wi = jnp.where(m > _NEG, wi_raw, i0)
        onehot = flat == wi
        stacked = jnp.concatenate(
            [jnp.where(onehot, c, _NEG) for c in (bx1, by1, bx2, by2)],
            axis=0)
        w = _redmax(stacked)
        wx1 = w[0:_B]
        wy1 = w[_B:2 * _B]
        wx2 = w[2 * _B:3 * _B]
        wy2 = w[3 * _B:]

        xx1 = jnp.maximum(wx1, bx1)
        yy1 = jnp.maximum(wy1, by1)
        xx2 = jnp.minimum(wx2, bx2)
        yy2 = jnp.minimum(wy2, by2)
        inter = jnp.maximum(xx2 - xx1, 0.0) * jnp.maximum(yy2 - yy1, 0.0)
        wa1 = jnp.maximum(wx2 - wx1, 0.0) * jnp.maximum(wy2 - wy1, 0.0)
        iou = inter / (wa1 + area - inter + 1e-8)

        new_active = jnp.where((iou > _IOU_THR) | onehot, _NEG, active)

        row = jnp.concatenate([wx1, wy1, wx2, wy2], axis=2)
        out_ref[t] = row.reshape(_B, 4)
        return new_active

    jax.lax.fori_loop(0, _POST_NMS, step, act0)


def kernel(rpn_scores, rpn_deltas, input_image):
    del input_image
    s = rpn_scores.reshape(_B, _ROWS, _LANES)
    tx = rpn_deltas[:, 0::4].reshape(_B, _ROWS, _LANES)
    ty = rpn_deltas[:, 1::4].reshape(_B, _ROWS, _LANES)
    tw = rpn_deltas[:, 2::4].reshape(_B, _ROWS, _LANES)
    th = rpn_deltas[:, 3::4].reshape(_B, _ROWS, _LANES)

    shp = jax.ShapeDtypeStruct((_B, _ROWS, _LANES), jnp.float32)
    shp_i = jax.ShapeDtypeStruct((_B, _ROWS, _LANES), jnp.int32)
    act, pos, x1, y1, x2, y2 = pl.pallas_call(
        _prep_body,
        out_shape=(shp, shp_i, shp, shp, shp, shp),
    )(s, tx, ty, tw, th)

    flat2 = lambda a: a.reshape(_B, _N)
    act_c, x1_c, y1_c, x2_c, y2_c = _compact_sc(
        flat2(act), flat2(pos), flat2(x1), flat2(y1), flat2(x2), flat2(y2))

    r3 = lambda a: a.reshape(_B, _CROWS, _LANES)
    out = pl.pallas_call(
        _scan_body,
        out_shape=jax.ShapeDtypeStruct((_POST_NMS, _B, 4), jnp.float32),
    )(r3(act_c), r3(x1_c), r3(y1_c), r3(x2_c), r3(y2_c))
    return out.transpose(1, 0, 2)


# speedup vs baseline: 1.5825x; 1.2715x over previous
import numpy as np

import jax
import jax.numpy as jnp
from jax.experimental import pallas as pl

_ANCHOR_SIZES = [64.0, 128.0, 256.0]
_ANCHOR_RATIOS = [float(np.sqrt(r)) for r in [0.5, 1.0, 2.0]]
_ANCHORS = np.array(
    [[s * r, s / r] for s in _ANCHOR_SIZES for r in _ANCHOR_RATIOS],
    dtype=np.float32,
)

_PRE_NMS = 6000
_POST_NMS = 300
_IOU_THR = 0.7
_NEG = -1e9
_BIG_IDX = 1 << 30
_ROWS = 72
_LANES = 128
_B = 4


def _iota2(shape, dim):
    return jax.lax.broadcasted_iota(jnp.int32, shape, dim)


def _redmax(x):
    return jnp.max(jnp.max(x, axis=1, keepdims=True), axis=2, keepdims=True)


def _redmin(x):
    return jnp.min(jnp.min(x, axis=1, keepdims=True), axis=2, keepdims=True)


def _redsum(x):
    return jnp.sum(jnp.sum(x, axis=1, keepdims=True), axis=2, keepdims=True)


def _nms_body(s_ref, tx_ref, ty_ref, tw_ref, th_ref, out_ref):
    s = s_ref[...]
    shape3 = s.shape

    flat = _iota2((_ROWS, _LANES), 0) * _LANES + _iota2((_ROWS, _LANES), 1)
    a_idx = flat >> 10
    hw = flat & 1023
    hh = (hw >> 5).astype(jnp.float32)
    ww = (hw & 31).astype(jnp.float32)

    wa = jnp.zeros((_ROWS, _LANES), jnp.float32)
    ha = jnp.zeros((_ROWS, _LANES), jnp.float32)
    for k in range(9):
        sel = a_idx == k
        wa = jnp.where(sel, jnp.float32(_ANCHORS[k, 0]), wa)
        ha = jnp.where(sel, jnp.float32(_ANCHORS[k, 1]), ha)

    px = (ww + 0.5) * 16.0
    py = (hh + 0.5) * 16.0
    ax1 = px - wa / 2.0
    ay1 = py - ha / 2.0
    cx = ax1 + 0.5 * wa
    cy = ay1 + 0.5 * ha

    ncx = cx + tx_ref[...] * wa
    ncy = cy + ty_ref[...] * ha
    nw = wa * jnp.exp(tw_ref[...])
    nh = ha * jnp.exp(th_ref[...])
    bx1 = jnp.clip(ncx - 0.5 * nw, 0.0, 511.0)
    by1 = jnp.clip(ncy - 0.5 * nh, 0.0, 511.0)
    bx2 = jnp.clip(ncx + 0.5 * nw, 0.0, 511.0)
    by2 = jnp.clip(ncy + 0.5 * nh, 0.0, 511.0)
    area = jnp.maximum(bx2 - bx1, 0.0) * jnp.maximum(by2 - by1, 0.0)

    s_bits = jax.lax.bitcast_convert_type(s, jnp.int32)

    def bs_step(_, carry):
        lo, hi = carry
        mid = (lo + hi) >> 1
        cnt = _redsum(jnp.where(s_bits > mid, 1.0, 0.0))
        pred = cnt < float(_PRE_NMS)
        lo2 = jnp.where(pred, lo, mid + 1)
        hi2 = jnp.where(pred, mid, hi)
        return lo2, hi2

    lo0 = jnp.zeros((_B, 1, 1), jnp.int32)
    hi0 = jnp.full((_B, 1, 1), 0x3F800000, jnp.int32)
    lo_f, _ = jax.lax.fori_loop(0, 31, bs_step, (lo0, hi0))
    thr = jax.lax.bitcast_convert_type(lo_f, jnp.float32)

    gt = s > thr
    eq = s == thr
    cg = _redsum(jnp.where(gt, 1.0, 0.0))
    r_adm = float(_PRE_NMS) - cg

    eqf = jnp.where(eq, 1.0, 0.0).reshape(_B * _ROWS, _LANES)
    lane_lt = jnp.where(
        _iota2((_LANES, _LANES), 0) < _iota2((_LANES, _LANES), 1), 1.0, 0.0)
    in_row = jax.lax.dot(eqf, lane_lt,
                         precision=jax.lax.Precision.HIGHEST,
                         preferred_element_type=jnp.float32)
    rowsum = jnp.sum(eqf, axis=1, keepdims=True)
    p = _iota2((_B * _ROWS, _B * _ROWS), 0)
    q = _iota2((_B * _ROWS, _B * _ROWS), 1)
    row_lt = jnp.where(((p // _ROWS) == (q // _ROWS)) & (q < p), 1.0, 0.0)
    row_off = jax.lax.dot(row_lt, rowsum,
                          precision=jax.lax.Precision.HIGHEST,
                          preferred_element_type=jnp.float32)
    prefix = (in_row + row_off).reshape(shape3)

    member = gt | (eq & (prefix < r_adm))
    active0 = jnp.where(member, s, _NEG)

    flat_f = flat.astype(jnp.float32)
    m0 = _redmax(s)
    i0 = _redmin(jnp.where(s == m0, flat_f, jnp.float32(_BIG_IDX)))

    def step(t, active):
        m = _redmax(active)
        wi_raw = _redmin(jnp.where(active == m, flat_f, jnp.float32(_BIG_IDX)))
        wi = jnp.where(m > _NEG, wi_raw, i0)
        onehot = flat_f == wi
        stacked = jnp.concatenate(
            [jnp.where(onehot, c, _NEG) for c in (bx1, by1, bx2, by2)],
            axis=0)
        w4 = _redmax(stacked)
        wx1 = w4[0:_B]
        wy1 = w4[_B:2 * _B]
        wx2 = w4[2 * _B:3 * _B]
        wy2 = w4[3 * _B:]

        xx1 = jnp.maximum(wx1, bx1)
        yy1 = jnp.maximum(wy1, by1)
        xx2 = jnp.minimum(wx2, bx2)
        yy2 = jnp.minimum(wy2, by2)
        inter = jnp.maximum(xx2 - xx1, 0.0) * jnp.maximum(yy2 - yy1, 0.0)
        wa1 = jnp.maximum(wx2 - wx1, 0.0) * jnp.maximum(wy2 - wy1, 0.0)
        iou = inter / (wa1 + area - inter + 1e-8)

        new_active = jnp.where(iou > _IOU_THR, _NEG, active)
        new_active = jnp.where(onehot, _NEG, new_active)

        row = jnp.concatenate([wx1, wy1, wx2, wy2], axis=2)
        out_ref[t] = row.reshape(_B, 4)
        return new_active

    jax.lax.fori_loop(0, _POST_NMS, step, active0)


def kernel(rpn_scores, rpn_deltas, input_image):
    del input_image
    s = rpn_scores.reshape(_B, _ROWS, _LANES)
    tx = rpn_deltas[:, 0::4].reshape(_B, _ROWS, _LANES)
    ty = rpn_deltas[:, 1::4].reshape(_B, _ROWS, _LANES)
    tw = rpn_deltas[:, 2::4].reshape(_B, _ROWS, _LANES)
    th = rpn_deltas[:, 3::4].reshape(_B, _ROWS, _LANES)
    out = pl.pallas_call(
        _nms_body,
        out_shape=jax.ShapeDtypeStruct((_POST_NMS, _B, 4), jnp.float32),
    )(s, tx, ty, tw, th)
    return out.transpose(1, 0, 2)
